# baseline (speedup 1.0000x reference)
import functools

import jax
import jax.numpy as jnp
from jax import lax
from jax.experimental import pallas as pl
from jax.experimental.pallas import tpu as pltpu
from jax.experimental.pallas import tpu_sc as plsc

OUT_D = 128
IDX_CHUNK = 128


def _make_sc_kernel(num_workers, b_per_w, n_chunks):
    mesh = plsc.VectorSubcoreMesh(core_axis_name="c", subcore_axis_name="s")
    num_cores = 2

    @functools.partial(
        pl.kernel,
        out_type=jax.ShapeDtypeStruct((num_workers * b_per_w, OUT_D),
                                      jnp.float32),
        mesh=mesh,
        scratch_types=[
            pltpu.VMEM((n_chunks, IDX_CHUNK), jnp.int32),
            pltpu.VMEM((b_per_w, OUT_D), jnp.float32),
            pltpu.SemaphoreType.DMA,
        ],
    )
    def emb_combine(x_hbm, comb_hbm, out_hbm, idx_v, g_v, sem):
        wid = lax.axis_index("s") * num_cores + lax.axis_index("c")
        base = wid * b_per_w
        pltpu.sync_copy(x_hbm.at[wid], idx_v)
        gathers = []
        for j in range(n_chunks):
            rows = pl.ds(j * IDX_CHUNK, IDX_CHUNK)
            gathers.append(
                pltpu.async_copy(comb_hbm.at[idx_v.at[j]], g_v.at[rows],
                                 sem))
        for c in gathers:
            c.wait()
        pltpu.sync_copy(g_v, out_hbm.at[pl.ds(base, b_per_w)])

    return emb_combine


def kernel(x, poi_table, loc_table):
    b = x.shape[0]
    info = plsc.get_sparse_core_info()
    num_workers = info.num_cores * info.num_subcores
    b_per_w = b // num_workers
    n_chunks = b_per_w // IDX_CHUNK
    x2 = x.astype(jnp.int32).reshape(num_workers, n_chunks, IDX_CHUNK)
    comb = jnp.concatenate([poi_table.T, loc_table.T], axis=0).T
    return _make_sc_kernel(num_workers, b_per_w, n_chunks)(x2, comb)


# speedup vs baseline: 1.0025x; 1.0025x over previous
import functools

import jax
import jax.numpy as jnp
from jax import lax
from jax.experimental import pallas as pl
from jax.experimental.pallas import tpu as pltpu
from jax.experimental.pallas import tpu_sc as plsc

OUT_D = 128
IDX_CHUNK = 128


def _make_sc_kernel(num_workers, b_per_w, n_chunks):
    mesh = plsc.VectorSubcoreMesh(core_axis_name="c", subcore_axis_name="s")
    num_cores = 2

    @functools.partial(
        pl.kernel,
        out_type=jax.ShapeDtypeStruct((num_workers * b_per_w, OUT_D),
                                      jnp.float32),
        mesh=mesh,
        scratch_types=[
            pltpu.VMEM((n_chunks, IDX_CHUNK), jnp.int32),
            pltpu.VMEM((b_per_w, OUT_D), jnp.float32),
            pltpu.SemaphoreType.DMA,
        ],
    )
    def emb_combine(x_hbm, comb_hbm, out_hbm, idx_v, g_v, sem):
        wid = lax.axis_index("s") * num_cores + lax.axis_index("c")
        base = wid * b_per_w
        pltpu.sync_copy(x_hbm.at[wid], idx_v)
        gathers = []
        for j in range(n_chunks):
            rows = pl.ds(j * IDX_CHUNK, IDX_CHUNK)
            gathers.append(
                pltpu.async_copy(comb_hbm.at[idx_v.at[j]], g_v.at[rows],
                                 sem))
        for c in gathers:
            c.wait()
        pltpu.sync_copy(g_v, out_hbm.at[pl.ds(base, b_per_w)])

    return emb_combine


def kernel(x, poi_table, loc_table):
    b = x.shape[0]
    info = plsc.get_sparse_core_info()
    num_workers = info.num_cores * info.num_subcores
    b_per_w = b // num_workers
    n_chunks = b_per_w // IDX_CHUNK
    x2 = x.astype(jnp.int32).reshape(num_workers, n_chunks, IDX_CHUNK)
    comb = jnp.concatenate([poi_table, loc_table], axis=1)
    return _make_sc_kernel(num_workers, b_per_w, n_chunks)(x2, comb)


# speedup vs baseline: 1.3203x; 1.3170x over previous
import functools

import jax
import jax.numpy as jnp
from jax import lax
from jax.experimental import pallas as pl
from jax.experimental.pallas import tpu as pltpu
from jax.experimental.pallas import tpu_sc as plsc

OUT_D = 128
IDX_CHUNK = 128


def _make_sc_kernel(num_workers, b_per_w, n_chunks):
    mesh = plsc.VectorSubcoreMesh(core_axis_name="c", subcore_axis_name="s")
    num_cores = 2

    @functools.partial(
        pl.kernel,
        out_type=jax.ShapeDtypeStruct((num_workers * b_per_w, OUT_D),
                                      jnp.float32),
        mesh=mesh,
        scratch_types=[
            pltpu.VMEM((n_chunks, IDX_CHUNK), jnp.int32),
            pltpu.VMEM((b_per_w, OUT_D), jnp.float32),
            pltpu.SemaphoreType.DMA,
        ],
    )
    def emb_combine(x_hbm, comb_hbm, out_hbm, idx_v, g_v, sem):
        wid = lax.axis_index("s") * num_cores + lax.axis_index("c")
        base = wid * b_per_w
        pltpu.sync_copy(x_hbm.at[wid], idx_v)
        gathers = []
        for j in range(n_chunks):
            rows = pl.ds(j * IDX_CHUNK, IDX_CHUNK)
            gathers.append(
                pltpu.async_copy(comb_hbm.at[idx_v.at[j]], g_v.at[rows],
                                 sem))
        for c in gathers:
            c.wait()
        pltpu.sync_copy(g_v, out_hbm.at[pl.ds(base, b_per_w)])

    return emb_combine


V_BLK = 2048


def _transpose_concat_kernel(poi_t_ref, loc_t_ref, out_ref):
    out_ref[:, :64] = poi_t_ref[...].T
    out_ref[:, 64:] = loc_t_ref[...].T


def _build_comb(poi_table, loc_table):
    v = poi_table.shape[0]
    grid = (v + V_BLK - 1) // V_BLK
    return pl.pallas_call(
        _transpose_concat_kernel,
        grid=(grid,),
        in_specs=[
            pl.BlockSpec((64, V_BLK), lambda i: (0, i)),
            pl.BlockSpec((64, V_BLK), lambda i: (0, i)),
        ],
        out_specs=pl.BlockSpec((V_BLK, OUT_D), lambda i: (i, 0)),
        out_shape=jax.ShapeDtypeStruct((v, OUT_D), jnp.float32),
    )(poi_table.T, loc_table.T)


def kernel(x, poi_table, loc_table):
    b = x.shape[0]
    info = plsc.get_sparse_core_info()
    num_workers = info.num_cores * info.num_subcores
    b_per_w = b // num_workers
    n_chunks = b_per_w // IDX_CHUNK
    x2 = x.astype(jnp.int32).reshape(num_workers, n_chunks, IDX_CHUNK)
    comb = _build_comb(poi_table, loc_table)
    return _make_sc_kernel(num_workers, b_per_w, n_chunks)(x2, comb)


# speedup vs baseline: 1.6468x; 1.2473x over previous
import functools

import jax
import jax.numpy as jnp
from jax import lax
from jax.experimental import pallas as pl
from jax.experimental.pallas import tpu as pltpu
from jax.experimental.pallas import tpu_sc as plsc

OUT_D = 128
IDX_CHUNK = 128


def _make_sc_kernel(num_workers, b_per_w, n_chunks):
    mesh = plsc.VectorSubcoreMesh(core_axis_name="c", subcore_axis_name="s")
    num_cores = 2

    @functools.partial(
        pl.kernel,
        out_type=jax.ShapeDtypeStruct((num_workers * b_per_w, OUT_D),
                                      jnp.float32),
        mesh=mesh,
        scratch_types=[
            pltpu.VMEM((n_chunks, IDX_CHUNK), jnp.int32),
            pltpu.VMEM((b_per_w, OUT_D), jnp.float32),
            pltpu.SemaphoreType.DMA,
        ],
    )
    def emb_combine(x_hbm, comb_hbm, out_hbm, idx_v, g_v, sem):
        wid = lax.axis_index("s") * num_cores + lax.axis_index("c")
        base = wid * b_per_w
        pltpu.sync_copy(x_hbm.at[wid], idx_v)
        gathers = []
        for j in range(n_chunks):
            rows = pl.ds(j * IDX_CHUNK, IDX_CHUNK)
            gathers.append(
                pltpu.async_copy(comb_hbm.at[idx_v.at[j]], g_v.at[rows],
                                 sem))
        for c in gathers:
            c.wait()
        pltpu.sync_copy(g_v, out_hbm.at[pl.ds(base, b_per_w)])

    return emb_combine


V_BLK = 8192


def _transpose_concat_kernel(poi_t_ref, loc_t_ref, out_ref):
    out_ref[:, :64] = poi_t_ref[...].T
    out_ref[:, 64:] = loc_t_ref[...].T


def _build_comb(poi_table, loc_table):
    v = poi_table.shape[0]
    grid = (v + V_BLK - 1) // V_BLK
    return pl.pallas_call(
        _transpose_concat_kernel,
        grid=(grid,),
        in_specs=[
            pl.BlockSpec((64, V_BLK), lambda i: (0, i)),
            pl.BlockSpec((64, V_BLK), lambda i: (0, i)),
        ],
        out_specs=pl.BlockSpec((V_BLK, OUT_D), lambda i: (i, 0)),
        out_shape=jax.ShapeDtypeStruct((v, OUT_D), jnp.float32),
    )(poi_table.T, loc_table.T)


def kernel(x, poi_table, loc_table):
    b = x.shape[0]
    info = plsc.get_sparse_core_info()
    num_workers = info.num_cores * info.num_subcores
    b_per_w = b // num_workers
    n_chunks = b_per_w // IDX_CHUNK
    x2 = x.astype(jnp.int32).reshape(num_workers, n_chunks, IDX_CHUNK)
    comb = _build_comb(poi_table, loc_table)
    return _make_sc_kernel(num_workers, b_per_w, n_chunks)(x2, comb)


# speedup vs baseline: 1.6996x; 1.0321x over previous
import functools

import jax
import jax.numpy as jnp
from jax import lax
from jax.experimental import pallas as pl
from jax.experimental.pallas import tpu as pltpu
from jax.experimental.pallas import tpu_sc as plsc

OUT_D = 128
IDX_CHUNK = 128


def _make_sc_kernel(num_workers, b_per_w, n_chunks):
    mesh = plsc.VectorSubcoreMesh(core_axis_name="c", subcore_axis_name="s")
    num_cores = 2

    @functools.partial(
        pl.kernel,
        out_type=jax.ShapeDtypeStruct((num_workers * b_per_w, OUT_D),
                                      jnp.float32),
        mesh=mesh,
        scratch_types=[
            pltpu.VMEM((n_chunks, IDX_CHUNK), jnp.int32),
            pltpu.VMEM((b_per_w, OUT_D), jnp.float32),
            pltpu.SemaphoreType.DMA,
        ],
    )
    def emb_combine(x_hbm, comb_hbm, out_hbm, idx_v, g_v, sem):
        wid = lax.axis_index("s") * num_cores + lax.axis_index("c")
        base = wid * b_per_w
        pltpu.sync_copy(x_hbm.at[wid], idx_v)
        gathers = []
        for j in range(n_chunks):
            rows = pl.ds(j * IDX_CHUNK, IDX_CHUNK)
            gathers.append(
                pltpu.async_copy(comb_hbm.at[idx_v.at[j]], g_v.at[rows],
                                 sem))
        for c in gathers:
            c.wait()
        pltpu.sync_copy(g_v, out_hbm.at[pl.ds(base, b_per_w)])

    return emb_combine


V_BLK = 12544


def _transpose_concat_kernel(poi_t_ref, loc_t_ref, out_ref):
    out_ref[:, :64] = poi_t_ref[...].T
    out_ref[:, 64:] = loc_t_ref[...].T


def _build_comb(poi_table, loc_table):
    v = poi_table.shape[0]
    grid = (v + V_BLK - 1) // V_BLK
    return pl.pallas_call(
        _transpose_concat_kernel,
        grid=(grid,),
        in_specs=[
            pl.BlockSpec((64, V_BLK), lambda i: (0, i)),
            pl.BlockSpec((64, V_BLK), lambda i: (0, i)),
        ],
        out_specs=pl.BlockSpec((V_BLK, OUT_D), lambda i: (i, 0)),
        out_shape=jax.ShapeDtypeStruct((v, OUT_D), jnp.float32),
    )(poi_table.T, loc_table.T)


def kernel(x, poi_table, loc_table):
    b = x.shape[0]
    info = plsc.get_sparse_core_info()
    num_workers = info.num_cores * info.num_subcores
    b_per_w = b // num_workers
    n_chunks = b_per_w // IDX_CHUNK
    x2 = x.astype(jnp.int32).reshape(num_workers, n_chunks, IDX_CHUNK)
    comb = _build_comb(poi_table, loc_table)
    return _make_sc_kernel(num_workers, b_per_w, n_chunks)(x2, comb)


# speedup vs baseline: 1.7070x; 1.0044x over previous
import functools

import jax
import jax.numpy as jnp
from jax import lax
from jax.experimental import pallas as pl
from jax.experimental.pallas import tpu as pltpu
from jax.experimental.pallas import tpu_sc as plsc

OUT_D = 128
IDX_CHUNK = 128


def _make_sc_kernel(num_workers, b_per_w, n_chunks):
    mesh = plsc.VectorSubcoreMesh(core_axis_name="c", subcore_axis_name="s")
    num_cores = 2

    @functools.partial(
        pl.kernel,
        out_type=jax.ShapeDtypeStruct((num_workers * b_per_w, OUT_D),
                                      jnp.float32),
        mesh=mesh,
        scratch_types=[
            pltpu.VMEM((n_chunks, IDX_CHUNK), jnp.int32),
            pltpu.VMEM((b_per_w, OUT_D), jnp.float32),
            pltpu.SemaphoreType.DMA,
        ],
    )
    def emb_combine(x_hbm, comb_hbm, out_hbm, idx_v, g_v, sem):
        wid = lax.axis_index("s") * num_cores + lax.axis_index("c")
        base = wid * b_per_w
        pltpu.sync_copy(x_hbm.at[wid], idx_v)
        gathers = []
        for j in range(n_chunks):
            rows = pl.ds(j * IDX_CHUNK, IDX_CHUNK)
            gathers.append(
                pltpu.async_copy(comb_hbm.at[idx_v.at[j]], g_v.at[rows],
                                 sem))
        for c in gathers:
            c.wait()
        pltpu.sync_copy(g_v, out_hbm.at[pl.ds(base, b_per_w)])

    return emb_combine


V_BLK = 12544


def _transpose_concat_kernel(poi_t_ref, loc_t_ref, out_ref):
    out_ref[:, :64] = poi_t_ref[...].T
    out_ref[:, 64:] = loc_t_ref[...].T


def _build_comb(poi_table, loc_table):
    v = poi_table.shape[0]
    grid = (v + V_BLK - 1) // V_BLK
    return pl.pallas_call(
        _transpose_concat_kernel,
        grid=(grid,),
        in_specs=[
            pl.BlockSpec((64, V_BLK), lambda i: (0, i)),
            pl.BlockSpec((64, V_BLK), lambda i: (0, i)),
        ],
        out_specs=pl.BlockSpec((V_BLK, OUT_D), lambda i: (i, 0)),
        out_shape=jax.ShapeDtypeStruct((v, OUT_D), jnp.float32),
        compiler_params=pltpu.CompilerParams(
            dimension_semantics=("parallel",)),
    )(poi_table.T, loc_table.T)


def kernel(x, poi_table, loc_table):
    b = x.shape[0]
    info = plsc.get_sparse_core_info()
    num_workers = info.num_cores * info.num_subcores
    b_per_w = b // num_workers
    n_chunks = b_per_w // IDX_CHUNK
    x2 = x.astype(jnp.int32).reshape(num_workers, n_chunks, IDX_CHUNK)
    comb = _build_comb(poi_table, loc_table)
    return _make_sc_kernel(num_workers, b_per_w, n_chunks)(x2, comb)
